# Initial kernel scaffold; baseline (speedup 1.0000x reference)
import jax
import jax.numpy as jnp
from jax.experimental import pallas as pl


def kernel(x, b1_w, b1_b, b2_w, b2_b, b3_w, b3_b, red_w, red_b, dw_w, dw_b, sep_scale, prior_w, prior_b, proj_w, proj_b, bn1_w1, bn1_b1, bn1_w2, bn1_b2, bn1_w3, bn1_b3, bn2_w1, bn2_b1, bn2_w2, bn2_b2, bn2_w3, bn2_b3, bn3_w1, bn3_b1, bn3_w2, bn3_b2, bn3_w3, bn3_b3, out_wb):
    raise NotImplementedError("write your pallas kernel here")


# speedup vs baseline: 1.0188x; 1.0188x over previous
import functools

import numpy as np
import jax
import jax.numpy as jnp
from jax.experimental import pallas as pl
from jax.experimental.pallas import tpu as pltpu


def _round_up(x, m):
    return ((x + m - 1) // m) * m


def _sigmoid(x):
    return 0.5 * (jnp.tanh(0.5 * x) + 1.0)


def _lpad(H, W, pb):
    return _round_up((H + 2 * pb) * (W + 2 * pb), 128)


def _border_mask_np(H, W, pb, L, reps):
    m = np.zeros((1, L), np.float32)
    Wp = W + 2 * pb
    for i in range(H):
        base = (i + pb) * Wp + pb
        m[0, base:base + W] = 1.0
    return np.tile(m, (1, reps))


def _shift(x, s, Lt):
    sh = (-s) % Lt
    if sh == 0:
        return x
    return pltpu.roll(x, sh, x.ndim - 1)


def _mm_kernel(a_ref, w_ref, b_ref, o_ref, *, act):
    acc = jnp.dot(a_ref[...], w_ref[...],
                  preferred_element_type=jnp.float32) + b_ref[...]
    if act:
        acc = jnp.maximum(acc, 0.0)
    o_ref[...] = acc


def _matmul(a, w, b, act, tile_m=2048):
    M, K = a.shape
    N = w.shape[1]
    b = jnp.asarray(b, jnp.float32).reshape(1, N)
    return pl.pallas_call(
        functools.partial(_mm_kernel, act=act),
        out_shape=jax.ShapeDtypeStruct((M, N), jnp.float32),
        grid=(M // tile_m,),
        in_specs=[pl.BlockSpec((tile_m, K), lambda i: (i, 0)),
                  pl.BlockSpec((K, N), lambda i: (0, 0)),
                  pl.BlockSpec((1, N), lambda i: (0, 0))],
        out_specs=pl.BlockSpec((tile_m, N), lambda i: (i, 0)),
        compiler_params=pltpu.CompilerParams(dimension_semantics=("parallel",)),
    )(a, w, b)


def _conv_s2(x, w, b):
    N, H, W, Ci = x.shape
    Co = w.shape[-1]
    x = x.astype(jnp.bfloat16)
    w = w.astype(jnp.bfloat16)
    xp = jnp.pad(x, ((0, 0), (1, 1), (1, 1), (0, 0)))
    Ho, Wo = H // 2, W // 2
    pats = [xp[:, i:i + 2 * Ho:2, j:j + 2 * Wo:2, :]
            for i in range(3) for j in range(3)]
    A = jnp.concatenate(pats, axis=-1).reshape(N * Ho * Wo, 9 * Ci)
    out = _matmul(A, w.reshape(9 * Ci, Co), b, act=True)
    return out.reshape(N, Ho, Wo, Co)


def _to_lanes(x_bchw, pb, L):
    B, C, H, W = x_bchw.shape
    Hp, Wp = H + 2 * pb, W + 2 * pb
    xp = jnp.pad(x_bchw, ((0, 0), (0, 0), (pb, pb), (pb, pb)))
    flat = jnp.pad(xp.reshape(B, C, Hp * Wp), ((0, 0), (0, 0), (0, L - Hp * Wp)))
    return flat.transpose(1, 0, 2).reshape(C, B * L)


def _from_lanes(xl, B, H, W, pb):
    C = xl.shape[0]
    L = xl.shape[1] // B
    Hp, Wp = H + 2 * pb, W + 2 * pb
    x = xl.reshape(C, B, L).transpose(1, 0, 2)[:, :, :Hp * Wp]
    return x.reshape(B, C, Hp, Wp)[:, :, pb:pb + H, pb:pb + W]


def _to_padded_b(x_bchw, pb, L):
    B, C, H, W = x_bchw.shape
    Hp, Wp = H + 2 * pb, W + 2 * pb
    xp = jnp.pad(x_bchw, ((0, 0), (0, 0), (pb, pb), (pb, pb)))
    return jnp.pad(xp.reshape(B, C, Hp * Wp), ((0, 0), (0, 0), (0, L - Hp * Wp)))


def _interp_matrix_np(out_size, in_size):
    A = np.zeros((out_size, in_size), np.float32)
    if out_size == 1 or in_size == 1:
        A[:, 0] = 1.0
        return A
    src = np.arange(out_size, dtype=np.float64) * (in_size - 1) / (out_size - 1)
    lo = np.clip(np.floor(src).astype(np.int64), 0, in_size - 2)
    w1 = (src - lo).astype(np.float32)
    w0 = 1.0 - w1
    A[np.arange(out_size), lo] += w0
    A[np.arange(out_size), lo + 1] += w1
    return A


def _upsample4(x4, hw_out):
    H2, W2 = hw_out
    Ah = jnp.asarray(_interp_matrix_np(H2, x4.shape[2]))
    Aw = jnp.asarray(_interp_matrix_np(W2, x4.shape[3]))
    return jnp.einsum('uh,bchw,vw->bcuv', Ah, x4, Aw)


def _sep_kernel(x_ref, redw_ref, redb_ref, dww_ref, dwb_ref, scale_ref,
                pw_ref, pbb_ref, projw_ref, projb_ref, mask_ref,
                intra_ref, inter_ref, staged, *, Wp, L, NB, K, inv_p, ic):
    Lt = NB * L
    mask = mask_ref[...]
    x = x_ref[...]
    t = 0
    for di in (-1, 0, 1):
        for dj in (-1, 0, 1):
            staged[pl.ds(t * ic, ic), :] = _shift(x, di * Wp + dj, Lt)
            t += 1
    v = jnp.maximum(jnp.dot(redw_ref[...], staged[...],
                            preferred_element_type=jnp.float32)
                    + redb_ref[...], 0.0) * mask

    dww = dww_ref[...]
    dwb = dwb_ref[...]

    def dw(z, w, b, step):
        acc = None
        for d in range(K):
            term = _shift(z, (d - K // 2) * step, Lt) * w[d]
            acc = term if acc is None else acc + term
        return (acc + b) * mask

    x1 = dw(dw(v, dww[0], dwb[0], Wp), dww[1], dwb[1], 1)
    x2 = dw(dw(v, dww[2], dwb[2], 1), dww[3], dwb[3], Wp)
    value = jnp.maximum((x1 + x2) * scale_ref[...], 0.0) * mask

    cam = _sigmoid(jnp.dot(pw_ref[...], value,
                           preferred_element_type=jnp.float32)
                   + pbb_ref[...])

    pw = projw_ref[...]
    pjb = projb_ref[...]
    for q in range(NB):
        sl = slice(q * L, (q + 1) * L)
        vq = value[:, sl]
        tmp = jnp.dot(vq, cam[:, sl], preferred_element_type=jnp.float32)
        rs = jnp.sum(vq, axis=1, keepdims=True)
        mq = mask[:, sl]
        intra_ref[:, sl] = jnp.maximum(
            jnp.dot(pw[0], tmp * inv_p, preferred_element_type=jnp.float32)
            + pjb[0], 0.0) * mq
        inter_ref[:, sl] = jnp.maximum(
            jnp.dot(pw[1], (rs - tmp) * inv_p, preferred_element_type=jnp.float32)
            + pjb[1], 0.0) * mq


def _sep_context(f3l, red_w, red_b, dw_w, dw_b, sep_scale, prior_wx, prior_bx,
                 proj_w, proj_b, mask_t, B, Wp, L, NB, K, inv_p):
    ic = f3l.shape[0]
    nc = red_w.shape[0]
    Lt = NB * L
    kern = functools.partial(_sep_kernel, Wp=Wp, L=L, NB=NB, K=K,
                             inv_p=inv_p, ic=ic)
    return pl.pallas_call(
        kern,
        out_shape=(jax.ShapeDtypeStruct((nc, B * L), jnp.float32),
                   jax.ShapeDtypeStruct((nc, B * L), jnp.float32)),
        grid=(B // NB,),
        in_specs=[
            pl.BlockSpec((ic, Lt), lambda j: (0, j)),
            pl.BlockSpec((nc, 9 * ic), lambda j: (0, 0)),
            pl.BlockSpec((nc, 1), lambda j: (0, 0)),
            pl.BlockSpec((4, K, nc, 1), lambda j: (0, 0, 0, 0)),
            pl.BlockSpec((4, nc, 1), lambda j: (0, 0, 0)),
            pl.BlockSpec((1, 1), lambda j: (0, 0)),
            pl.BlockSpec((L, nc), lambda j: (0, 0)),
            pl.BlockSpec((L, 1), lambda j: (0, 0)),
            pl.BlockSpec((2, nc, nc), lambda j: (0, 0, 0)),
            pl.BlockSpec((2, nc, 1), lambda j: (0, 0, 0)),
            pl.BlockSpec((1, Lt), lambda j: (0, 0)),
        ],
        out_specs=(pl.BlockSpec((nc, Lt), lambda j: (0, j)),
                   pl.BlockSpec((nc, Lt), lambda j: (0, j))),
        scratch_shapes=[pltpu.VMEM((9 * ic, Lt), jnp.float32)],
        compiler_params=pltpu.CompilerParams(dimension_semantics=("parallel",)),
    )(f3l, red_w, red_b, dw_w, dw_b, sep_scale, prior_wx, prior_bx,
      proj_w, proj_b, mask_t)


def _bnc_kernel(x_ref, w1_ref, b1_ref, w2_ref, b2_ref, w3_ref, b3_ref,
                mask_ref, o_ref, st1, st2, *, Wp, L, NB, ctot, nc):
    Lt = NB * L
    mask = mask_ref[...]
    x = x_ref[...].astype(jnp.bfloat16)
    t = 0
    for di in (-1, 0, 1):
        for dj in (-1, 0, 1):
            st1[pl.ds(t * ctot, ctot), :] = _shift(x, di * Wp + dj, Lt)
            t += 1
    h = jnp.maximum(jnp.dot(w1_ref[...], st1[...],
                            preferred_element_type=jnp.float32)
                    + b1_ref[...], 0.0) * mask
    h16 = h.astype(jnp.bfloat16)
    t = 0
    for di in (-1, 0, 1):
        for dj in (-1, 0, 1):
            st2[pl.ds(t * nc, nc), :] = _shift(h16, di * Wp + dj, Lt)
            t += 1
    h = jnp.maximum(jnp.dot(w2_ref[...], st2[...],
                            preferred_element_type=jnp.float32)
                    + b2_ref[...], 0.0) * mask
    h16 = h.astype(jnp.bfloat16)
    t = 0
    for di in (-1, 0, 1):
        for dj in (-1, 0, 1):
            st2[pl.ds(t * nc, nc), :] = _shift(h16, di * Wp + dj, Lt)
            t += 1
    o_ref[...] = jnp.maximum(jnp.dot(w3_ref[...], st2[...],
                                     preferred_element_type=jnp.float32)
                             + b3_ref[...], 0.0) * mask


def _bottleneck_cat(xl, plist, mask_t, B, Wp, L, NB):
    ctot = xl.shape[0]
    (w1, b1), (w2, b2), (w3, b3) = plist
    nc = w1.shape[0]
    Lt = NB * L
    kern = functools.partial(_bnc_kernel, Wp=Wp, L=L, NB=NB, ctot=ctot, nc=nc)
    return pl.pallas_call(
        kern,
        out_shape=jax.ShapeDtypeStruct((1, B * L), jnp.float32),
        grid=(B // NB,),
        in_specs=[
            pl.BlockSpec((ctot, Lt), lambda j: (0, j)),
            pl.BlockSpec((nc, 9 * ctot), lambda j: (0, 0)),
            pl.BlockSpec((nc, 1), lambda j: (0, 0)),
            pl.BlockSpec((nc, 9 * nc), lambda j: (0, 0)),
            pl.BlockSpec((nc, 1), lambda j: (0, 0)),
            pl.BlockSpec((1, 9 * nc), lambda j: (0, 0)),
            pl.BlockSpec((1, 1), lambda j: (0, 0)),
            pl.BlockSpec((1, Lt), lambda j: (0, 0)),
        ],
        out_specs=pl.BlockSpec((1, Lt), lambda j: (0, j)),
        scratch_shapes=[pltpu.VMEM((9 * ctot, Lt), jnp.bfloat16),
                        pltpu.VMEM((9 * nc, Lt), jnp.bfloat16)],
        compiler_params=pltpu.CompilerParams(dimension_semantics=("parallel",)),
    )(xl.astype(jnp.bfloat16), w1.astype(jnp.bfloat16), b1,
      w2.astype(jnp.bfloat16), b2, w3.astype(jnp.bfloat16), b3, mask_t)


def _bn3_kernel(g1_ref, g2_ref, w1_ref, b1_ref, w2_ref, b2_ref, w3_ref, b3_ref,
                mask_ref, o_ref, st1, st2, *, Wp, L, c1, c2, nc):
    mask = mask_ref[...]
    a = g1_ref[0].astype(jnp.bfloat16)
    c = g2_ref[0].astype(jnp.bfloat16)
    ctot = c1 + c2
    t = 0
    for di in (-1, 0, 1):
        for dj in (-1, 0, 1):
            s = di * Wp + dj
            st1[pl.ds(t * ctot, c1), :] = _shift(a, s, L)
            st1[pl.ds(t * ctot + c1, c2), :] = _shift(c, s, L)
            t += 1
    h = jnp.maximum(jnp.dot(w1_ref[...], st1[...],
                            preferred_element_type=jnp.float32)
                    + b1_ref[...], 0.0) * mask
    h16 = h.astype(jnp.bfloat16)
    t = 0
    for di in (-1, 0, 1):
        for dj in (-1, 0, 1):
            st2[pl.ds(t * nc, nc), :] = _shift(h16, di * Wp + dj, L)
            t += 1
    h = jnp.maximum(jnp.dot(w2_ref[...], st2[...],
                            preferred_element_type=jnp.float32)
                    + b2_ref[...], 0.0) * mask
    h16 = h.astype(jnp.bfloat16)
    t = 0
    for di in (-1, 0, 1):
        for dj in (-1, 0, 1):
            st2[pl.ds(t * nc, nc), :] = _shift(h16, di * Wp + dj, L)
            t += 1
    o_ref[...] = (jnp.maximum(jnp.dot(w3_ref[...], st2[...],
                                      preferred_element_type=jnp.float32)
                              + b3_ref[...], 0.0) * mask)[None]


def _bottleneck3(g1, g2, plist, mask_np, B, Wp, L):
    c1, c2 = g1.shape[1], g2.shape[1]
    (w1, b1), (w2, b2), (w3, b3) = plist
    nc = w1.shape[0]
    ctot = c1 + c2
    kern = functools.partial(_bn3_kernel, Wp=Wp, L=L, c1=c1, c2=c2, nc=nc)
    return pl.pallas_call(
        kern,
        out_shape=jax.ShapeDtypeStruct((B, 1, L), jnp.float32),
        grid=(B,),
        in_specs=[
            pl.BlockSpec((1, c1, L), lambda b: (b, 0, 0)),
            pl.BlockSpec((1, c2, L), lambda b: (b, 0, 0)),
            pl.BlockSpec((nc, 9 * ctot), lambda b: (0, 0)),
            pl.BlockSpec((nc, 1), lambda b: (0, 0)),
            pl.BlockSpec((nc, 9 * nc), lambda b: (0, 0)),
            pl.BlockSpec((nc, 1), lambda b: (0, 0)),
            pl.BlockSpec((1, 9 * nc), lambda b: (0, 0)),
            pl.BlockSpec((1, 1), lambda b: (0, 0)),
            pl.BlockSpec((1, L), lambda b: (0, 0)),
        ],
        out_specs=pl.BlockSpec((1, 1, L), lambda b: (b, 0, 0)),
        scratch_shapes=[pltpu.VMEM((9 * ctot, L), jnp.bfloat16),
                        pltpu.VMEM((9 * nc, L), jnp.bfloat16)],
        compiler_params=pltpu.CompilerParams(dimension_semantics=("parallel",)),
    )(g1, g2, w1.astype(jnp.bfloat16), b1, w2.astype(jnp.bfloat16), b2,
      w3.astype(jnp.bfloat16), b3, jnp.asarray(mask_np))


def _tail_kernel(u1_ref, u2_ref, o3_ref, w_ref, out_ref):
    w = w_ref[...]
    y = u1_ref[:, 0] * w[0] + u2_ref[:, 0] * w[1] + o3_ref[:, 0] * w[2] + w[3]
    out_ref[...] = _sigmoid(y)[:, None]


def _tail(u1, u2, o3, wb, NB=16):
    B, _, L = u1.shape
    return pl.pallas_call(
        _tail_kernel,
        out_shape=jax.ShapeDtypeStruct((B, 1, L), jnp.float32),
        grid=(B // NB,),
        in_specs=[pl.BlockSpec((NB, 1, L), lambda b: (b, 0, 0)),
                  pl.BlockSpec((NB, 1, L), lambda b: (b, 0, 0)),
                  pl.BlockSpec((NB, 1, L), lambda b: (b, 0, 0)),
                  pl.BlockSpec((4, 1, 1), lambda b: (0, 0, 0))],
        out_specs=pl.BlockSpec((NB, 1, L), lambda b: (b, 0, 0)),
        compiler_params=pltpu.CompilerParams(dimension_semantics=("parallel",)),
    )(u1, u2, o3, wb)


def kernel(x, b1_w, b1_b, b2_w, b2_b, b3_w, b3_b, red_w, red_b, dw_w, dw_b,
           sep_scale, prior_w, prior_b, proj_w, proj_b,
           bn1_w1, bn1_b1, bn1_w2, bn1_b2, bn1_w3, bn1_b3,
           bn2_w1, bn2_b1, bn2_w2, bn2_b2, bn2_w3, bn2_b3,
           bn3_w1, bn3_b1, bn3_w2, bn3_b2, bn3_w3, bn3_b3, out_wb):
    K = 3
    pb = 1
    xh = jnp.transpose(x, (0, 2, 3, 1)).astype(jnp.float32)

    f1 = _conv_s2(xh, b1_w, b1_b)
    f2 = _conv_s2(f1, b2_w, b2_b)
    f3 = _conv_s2(f2, b3_w, b3_b)

    B, h3, w3, ic = f3.shape
    _, h2, w2, nc = f2.shape
    _, h1, w1, _ = f1.shape
    L3, L2, L1 = _lpad(h3, w3, pb), _lpad(h2, w2, pb), _lpad(h1, w1, pb)
    Wp3, Wp2, Wp1 = w3 + 2 * pb, w2 + 2 * pb, w1 + 2 * pb
    NB3, NB2 = 16, 8

    f3l = _to_lanes(jnp.transpose(f3, (0, 3, 1, 2)), pb, L3)
    f2l = _to_lanes(jnp.transpose(f2, (0, 3, 1, 2)), pb, L2)
    f1b = _to_padded_b(jnp.transpose(f1, (0, 3, 1, 2)), pb, L1)

    mask3_t = jnp.asarray(_border_mask_np(h3, w3, pb, L3, NB3))
    mask2_t = jnp.asarray(_border_mask_np(h2, w2, pb, L2, NB2))
    mask1 = _border_mask_np(h1, w1, pb, L1, 1)

    pos = ((np.arange(h3)[:, None] + pb) * Wp3
           + (np.arange(w3)[None, :] + pb)).reshape(-1)
    prior_wx = jnp.zeros((L3, nc), jnp.float32).at[pos].set(prior_w)
    prior_bx = jnp.zeros((L3, 1), jnp.float32).at[pos].set(
        prior_b.reshape(-1, 1))

    intra, inter = _sep_context(f3l, red_w, red_b, dw_w, dw_b, sep_scale,
                                prior_wx, prior_bx, proj_w, proj_b, mask3_t,
                                B, Wp3, L3, NB3, K, 1.0 / float(h3 * w3))

    cal_out1 = jnp.concatenate([f3l, intra, inter], axis=0)
    co1_4 = _from_lanes(cal_out1, B, h3, w3, pb)
    cal_up1 = _to_lanes(_upsample4(co1_4, (h2, w2)), pb, L2)
    cal_out2 = jnp.concatenate([f2l, cal_up1], axis=0)
    co2_4 = _from_lanes(cal_out2, B, h2, w2, pb)
    cal_up2 = _to_padded_b(_upsample4(co2_4, (h1, w1)), pb, L1)

    out1 = _bottleneck_cat(cal_out1,
                           [(bn1_w1, bn1_b1), (bn1_w2, bn1_b2),
                            (bn1_w3, bn1_b3)], mask3_t, B, Wp3, L3, NB3)
    out2 = _bottleneck_cat(cal_out2,
                           [(bn2_w1, bn2_b1), (bn2_w2, bn2_b2),
                            (bn2_w3, bn2_b3)], mask2_t, B, Wp2, L2, NB2)
    out3 = _bottleneck3(f1b, cal_up2,
                        [(bn3_w1, bn3_b1), (bn3_w2, bn3_b2), (bn3_w3, bn3_b3)],
                        mask1, B, Wp1, L1)

    u1 = _to_padded_b(_upsample4(_from_lanes(out1, B, h3, w3, pb),
                                 (h1, w1)), pb, L1)
    u2 = _to_padded_b(_upsample4(_from_lanes(out2, B, h2, w2, pb),
                                 (h1, w1)), pb, L1)

    outp = _tail(u1, u2, out3, out_wb)
    Hp1, Wp1f = h1 + 2 * pb, w1 + 2 * pb
    o = outp[:, :, :Hp1 * Wp1f].reshape(B, 1, Hp1, Wp1f)
    return o[:, :, pb:pb + h1, pb:pb + w1]
